# baseline (speedup 1.0000x reference)
import functools

import jax
import jax.numpy as jnp
from jax import lax
from jax.experimental import pallas as pl
from jax.experimental.pallas import tpu as pltpu
from jax.experimental.pallas import tpu_sc as plsc

NUM_GATES = 1000
GPAD = 1024
BATCH = 16384
DIM = 128
LANES = 16

NC = 2
NS = 16
NW = NC * NS

B_PER_W = BATCH // NW
CHUNK = 128
NCHUNK = B_PER_W // CHUNK
NBUF = 3
G_PER_S = GPAD // NS


def _sc_gate_mul(x, idx, gates):
    mesh = plsc.VectorSubcoreMesh(core_axis_name="c", subcore_axis_name="s")

    @functools.partial(
        pl.kernel,
        mesh=mesh,
        out_type=jax.ShapeDtypeStruct((BATCH, DIM), jnp.float32),
        scratch_types=[
            pltpu.VMEM((NCHUNK, CHUNK), jnp.int32),
            pltpu.VMEM((NBUF, CHUNK, DIM), jnp.float32),
            pltpu.VMEM((NBUF, CHUNK, DIM), jnp.float32),
            pltpu.VMEM_SHARED((GPAD, DIM), jnp.float32),
            pltpu.SemaphoreType.DMA((NBUF,)),
            pltpu.SemaphoreType.DMA((NBUF,)),
            pltpu.SemaphoreType.DMA((NBUF,)),
        ],
    )
    def k(x_hbm, idx_hbm, gates_hbm, out_hbm, idx_v, g_v, x_v, gates_sp,
          gsem, xsem, osem):
        sid = lax.axis_index("s")
        wid = sid * NC + lax.axis_index("c")
        base = wid * B_PER_W

        pltpu.sync_copy(gates_hbm.at[pl.ds(sid * G_PER_S, G_PER_S)],
                        gates_sp.at[pl.ds(sid * G_PER_S, G_PER_S)])
        pltpu.sync_copy(idx_hbm.at[pl.ds(wid * NCHUNK, NCHUNK)], idx_v)
        plsc.subcore_barrier()

        def fetch(c):
            b = c % NBUF
            pltpu.async_copy(gates_sp.at[idx_v.at[c]], g_v.at[b],
                             gsem.at[b])
            pltpu.async_copy(x_hbm.at[pl.ds(base + c * CHUNK, CHUNK)],
                             x_v.at[b], xsem.at[b])

        fetch(0)
        for c in range(NCHUNK):
            b = c % NBUF
            if c + 1 < NCHUNK:
                if c + 1 >= NBUF:
                    pltpu.make_async_copy(
                        x_v.at[(c + 1) % NBUF],
                        out_hbm.at[pl.ds(base + (c + 1 - NBUF) * CHUNK,
                                         CHUNK)],
                        osem.at[(c + 1) % NBUF],
                    ).wait()
                fetch(c + 1)
            pltpu.make_async_copy(gates_sp.at[idx_v.at[c]], g_v.at[b],
                                  gsem.at[b]).wait()
            pltpu.make_async_copy(x_hbm.at[pl.ds(base + c * CHUNK, CHUNK)],
                                  x_v.at[b], xsem.at[b]).wait()

            @plsc.parallel_loop(0, CHUNK, step=1, unroll=8)
            def row_body(r):
                for j in range(DIM // LANES):
                    sl = pl.ds(j * LANES, LANES)
                    x_v[b, r, sl] = x_v[b, r, sl] * g_v[b, r, sl]

            pltpu.async_copy(x_v.at[b],
                             out_hbm.at[pl.ds(base + c * CHUNK, CHUNK)],
                             osem.at[b])

        for c in range(max(0, NCHUNK - NBUF), NCHUNK):
            b = c % NBUF
            pltpu.make_async_copy(
                x_v.at[b],
                out_hbm.at[pl.ds(base + c * CHUNK, CHUNK)],
                osem.at[b],
            ).wait()

    return k(x, idx, gates)


def kernel(input, gate_index, gates):
    idx = gate_index.astype(jnp.int32).reshape(NW * NCHUNK, CHUNK)
    gates_pad = jnp.concatenate(
        [gates, jnp.zeros((GPAD - NUM_GATES, DIM), gates.dtype)], axis=0)
    return _sc_gate_mul(input, idx, gates_pad)


# speedup vs baseline: 1.0378x; 1.0378x over previous
import functools

import jax
import jax.numpy as jnp
from jax import lax
from jax.experimental import pallas as pl
from jax.experimental.pallas import tpu as pltpu
from jax.experimental.pallas import tpu_sc as plsc

NUM_GATES = 1000
GPAD = 1024
BATCH = 16384
DIM = 128
LANES = 16

NC = 2
NS = 16
NW = NC * NS

B_PER_W = BATCH // NW
CHUNK = 128
NCHUNK = B_PER_W // CHUNK
NBUF = 3
G_PER_S = GPAD // NS


def _sc_gate_mul(x, idx, gates):
    mesh = plsc.VectorSubcoreMesh(core_axis_name="c", subcore_axis_name="s")

    @functools.partial(
        pl.kernel,
        mesh=mesh,
        out_type=jax.ShapeDtypeStruct((BATCH, DIM), jnp.float32),
        scratch_types=[
            pltpu.VMEM((NCHUNK, CHUNK), jnp.int32),
            pltpu.VMEM((NBUF, CHUNK, DIM), jnp.float32),
            pltpu.VMEM((NBUF, CHUNK, DIM), jnp.float32),
            pltpu.VMEM_SHARED((GPAD, DIM), jnp.float32),
            pltpu.SemaphoreType.DMA((NBUF,)),
            pltpu.SemaphoreType.DMA((NBUF,)),
            pltpu.SemaphoreType.DMA((NBUF,)),
        ],
    )
    def k(x_hbm, idx_hbm, gates_hbm, out_hbm, idx_v, g_v, x_v, gates_sp,
          gsem, xsem, osem):
        sid = lax.axis_index("s")
        wid = sid * NC + lax.axis_index("c")
        base = wid * B_PER_W

        pltpu.sync_copy(gates_hbm.at[pl.ds(sid * G_PER_S, G_PER_S)],
                        gates_sp.at[pl.ds(sid * G_PER_S, G_PER_S)])
        pltpu.sync_copy(idx_hbm.at[pl.ds(wid * NCHUNK, NCHUNK)], idx_v)
        plsc.subcore_barrier()

        def fetch(c):
            b = c % NBUF
            pltpu.async_copy(gates_sp.at[idx_v.at[c]], g_v.at[b],
                             gsem.at[b])
            pltpu.async_copy(x_hbm.at[pl.ds(base + c * CHUNK, CHUNK)],
                             x_v.at[b], xsem.at[b])

        fetch(0)
        for c in range(NCHUNK):
            b = c % NBUF
            if c + 1 < NCHUNK:
                if c + 1 >= NBUF:
                    pltpu.make_async_copy(
                        x_v.at[(c + 1) % NBUF],
                        out_hbm.at[pl.ds(base + (c + 1 - NBUF) * CHUNK,
                                         CHUNK)],
                        osem.at[(c + 1) % NBUF],
                    ).wait()
                fetch(c + 1)
            pltpu.make_async_copy(gates_sp.at[idx_v.at[c]], g_v.at[b],
                                  gsem.at[b]).wait()
            pltpu.make_async_copy(x_hbm.at[pl.ds(base + c * CHUNK, CHUNK)],
                                  x_v.at[b], xsem.at[b]).wait()

            @plsc.parallel_loop(0, CHUNK, step=1, unroll=4)
            def row_body(r):
                for j in range(DIM // LANES):
                    sl = pl.ds(j * LANES, LANES)
                    x_v[b, r, sl] = x_v[b, r, sl] * g_v[b, r, sl]

            pltpu.async_copy(x_v.at[b],
                             out_hbm.at[pl.ds(base + c * CHUNK, CHUNK)],
                             osem.at[b])

        for c in range(max(0, NCHUNK - NBUF), NCHUNK):
            b = c % NBUF
            pltpu.make_async_copy(
                x_v.at[b],
                out_hbm.at[pl.ds(base + c * CHUNK, CHUNK)],
                osem.at[b],
            ).wait()

    return k(x, idx, gates)


def kernel(input, gate_index, gates):
    idx = gate_index.astype(jnp.int32).reshape(NW * NCHUNK, CHUNK)
    gates_pad = jnp.concatenate(
        [gates, jnp.zeros((GPAD - NUM_GATES, DIM), gates.dtype)], axis=0)
    return _sc_gate_mul(input, idx, gates_pad)
